# baseline (speedup 1.0000x reference)
import functools

import jax
import jax.numpy as jnp
from jax import lax
from jax.experimental import pallas as pl
from jax.experimental.pallas import tpu as pltpu
from jax.experimental.pallas import tpu_sc as plsc

_B, _H, _D = 4096, 200, 64
_N = _B * _H
_NC, _NS = 2, 16
_NW = _NC * _NS
_PER_W = _N // _NW
_CHUNK = 128
_NCH = _PER_W // _CHUNK
_NBUF = 4
_NGRP = _NCH // _NBUF


def _build():
  mesh = plsc.VectorSubcoreMesh(core_axis_name="c", subcore_axis_name="s")
  scratch = [
      pltpu.VMEM((_NCH, _CHUNK), jnp.int32),
      pltpu.VMEM((_NBUF, _CHUNK, _D), jnp.float32),
  ] + [pltpu.SemaphoreType.DMA] * (2 * _NBUF)

  @functools.partial(
      pl.kernel,
      out_type=jax.ShapeDtypeStruct((_N, _D), jnp.float32),
      mesh=mesh,
      scratch_types=scratch,
      compiler_params=pltpu.CompilerParams(use_tc_tiling_on_sc=False),
  )
  def body(idx_hbm, table_hbm, out_hbm, idx_v, rows, *sems):
    gsem = sems[:_NBUF]
    osem = sems[_NBUF:]
    wid = lax.axis_index("s") * _NC + lax.axis_index("c")
    base = wid * _PER_W
    pltpu.sync_copy(idx_hbm.at[wid], idx_v)

    def start_gather(ch, b):
      pltpu.async_copy(table_hbm.at[idx_v.at[ch]], rows.at[b], gsem[b])

    def wait_gather(b):
      pltpu.make_async_copy(
          table_hbm.at[idx_v.at[0]], rows.at[b], gsem[b]).wait()

    def start_out(ch, b):
      pltpu.async_copy(
          rows.at[b], out_hbm.at[pl.ds(base + ch * _CHUNK, _CHUNK)], osem[b])

    def wait_out(b):
      pltpu.make_async_copy(
          rows.at[b], out_hbm.at[pl.ds(base, _CHUNK)], osem[b]).wait()

    def ring(g, issue_next):
      for b in range(_NBUF):
        ch = g * _NBUF + b
        wait_gather(b)
        start_out(ch, b)
        wait_out(b)
        if issue_next:
          start_gather(ch + _NBUF, b)

    for b in range(_NBUF):
      start_gather(b, b)

    def loop_body(g, carry):
      ring(g, True)
      return carry

    lax.fori_loop(0, _NGRP - 1, loop_body, 0)
    ring(_NGRP - 1, False)

  return body


_gather = _build()


@jax.jit
def kernel(input, weight):
  idx = input.reshape(_NW, _NCH, _CHUNK).astype(jnp.int32)
  out = _gather(idx, weight)
  return out.reshape(_B, _H, _D)


# speedup vs baseline: 1.2174x; 1.2174x over previous
import functools

import jax
import jax.numpy as jnp
from jax import lax
from jax.experimental import pallas as pl
from jax.experimental.pallas import tpu as pltpu
from jax.experimental.pallas import tpu_sc as plsc

_B, _H, _D = 4096, 200, 64
_V = 1000000
_N = _B * _H
_NC, _NS = 2, 16
_NW = _NC * _NS
_PER_W = _N // _NW
_CHUNK = 128
_NCH = _PER_W // _CHUNK
_NBUF = 3


def _build():
  mesh = plsc.VectorSubcoreMesh(core_axis_name="c", subcore_axis_name="s")
  scratch = [
      pltpu.VMEM((_NCH, _CHUNK), jnp.int32),
      pltpu.VMEM((_NBUF, _CHUNK, 2 * _D), jnp.float32),
      pltpu.VMEM((_NBUF, _CHUNK, _D), jnp.float32),
  ] + [pltpu.SemaphoreType.DMA] * (2 * _NBUF)

  @functools.partial(
      pl.kernel,
      out_type=jax.ShapeDtypeStruct((_N, _D), jnp.float32),
      mesh=mesh,
      scratch_types=scratch,
      compiler_params=pltpu.CompilerParams(use_tc_tiling_on_sc=True),
  )
  def body(idx_hbm, tbl_hbm, out_hbm, idx_v, rows, rows64, *sems):
    gsem = sems[:_NBUF]
    osem = sems[_NBUF:]
    wid = lax.axis_index("s") * _NC + lax.axis_index("c")
    base = wid * _PER_W
    pltpu.sync_copy(idx_hbm.at[wid], idx_v)

    def start_gather(ch, b):
      pltpu.async_copy(tbl_hbm.at[idx_v.at[ch]], rows.at[b], gsem[b])

    def wait_gather(b):
      pltpu.make_async_copy(
          tbl_hbm.at[idx_v.at[0]], rows.at[b], gsem[b]).wait()

    def compact(b):
      def row(t, carry):
        for kk in range(_D // 16):
          rows64[b, t, pl.ds(kk * 16, 16)] = rows[b, t, pl.ds(kk * 16, 16)]
        return carry
      lax.fori_loop(0, _CHUNK, row, 0)

    def start_out(ch, b):
      pltpu.async_copy(
          rows64.at[b], out_hbm.at[pl.ds(base + ch * _CHUNK, _CHUNK)], osem[b])

    def wait_out(b):
      pltpu.make_async_copy(
          rows64.at[b], out_hbm.at[pl.ds(base, _CHUNK)], osem[b]).wait()

    for b in range(_NBUF):
      start_gather(b, b)

    def loop_body(g, carry):
      for b in range(_NBUF):
        ch = g * _NBUF + b
        wait_gather(b)
        compact(b)
        start_out(ch, b)
        wait_out(b)
        start_gather(ch + _NBUF, b)
      return carry

    _NGRP = _NCH // _NBUF
    lax.fori_loop(0, _NGRP - 1, loop_body, 0)
    for b in range(_NBUF):
      ch = (_NGRP - 1) * _NBUF + b
      wait_gather(b)
      compact(b)
      start_out(ch, b)
      wait_out(b)
    for ch in range(_NGRP * _NBUF, _NCH):
      b = ch % _NBUF
      start_gather(ch, b)
      wait_gather(b)
      compact(b)
      start_out(ch, b)
      wait_out(b)

  return body


_gather = _build()


@jax.jit
def kernel(input, weight):
  tbl = jnp.pad(weight, ((0, 0), (0, _D)))
  idx = input.reshape(_NW, _NCH, _CHUNK).astype(jnp.int32)
  out = _gather(idx, tbl)
  return out.reshape(_B, _H, _D)


# speedup vs baseline: 1.2208x; 1.0028x over previous
import functools

import jax
import jax.numpy as jnp
from jax import lax
from jax.experimental import pallas as pl
from jax.experimental.pallas import tpu as pltpu
from jax.experimental.pallas import tpu_sc as plsc

_B, _H, _D = 4096, 200, 64
_V = 1000000
_N = _B * _H
_NC, _NS = 2, 16
_NW = _NC * _NS
_PER_W = _N // _NW
_CHUNK = 128
_NCH = _PER_W // _CHUNK
_NBUF = 3


def _build():
  mesh = plsc.VectorSubcoreMesh(core_axis_name="c", subcore_axis_name="s")
  scratch = [
      pltpu.VMEM((_NCH, _CHUNK), jnp.int32),
      pltpu.VMEM((_NBUF, _CHUNK, 2 * _D), jnp.float32),
      pltpu.VMEM((_NBUF, _CHUNK, _D), jnp.float32),
  ] + [pltpu.SemaphoreType.DMA] * (2 * _NBUF)

  @functools.partial(
      pl.kernel,
      out_type=jax.ShapeDtypeStruct((_N, _D), jnp.float32),
      mesh=mesh,
      scratch_types=scratch,
      compiler_params=pltpu.CompilerParams(use_tc_tiling_on_sc=True),
  )
  def body(idx_hbm, tbl_hbm, out_hbm, idx_v, rows, rows64, *sems):
    gsem = sems[:_NBUF]
    osem = sems[_NBUF:]
    wid = lax.axis_index("s") * _NC + lax.axis_index("c")
    base = wid * _PER_W
    pltpu.sync_copy(idx_hbm.at[wid], idx_v)

    def start_gather(ch, b):
      pltpu.async_copy(tbl_hbm.at[idx_v.at[ch]], rows.at[b], gsem[b])

    def wait_gather(b):
      pltpu.make_async_copy(
          tbl_hbm.at[idx_v.at[0]], rows.at[b], gsem[b]).wait()

    def compact(b):
      def row(t, carry):
        for kk in range(_D // 16):
          rows64[b, t, pl.ds(kk * 16, 16)] = rows[b, t, pl.ds(kk * 16, 16)]
        return carry
      lax.fori_loop(0, _CHUNK, row, 0)

    def start_out(ch, b):
      pltpu.async_copy(
          rows64.at[b], out_hbm.at[pl.ds(base + ch * _CHUNK, _CHUNK)], osem[b])

    def wait_out(b):
      pltpu.make_async_copy(
          rows64.at[b], out_hbm.at[pl.ds(base, _CHUNK)], osem[b]).wait()

    _NGRP = _NCH // _NBUF

    for b in range(_NBUF):
      start_gather(b, b)
    for b in range(_NBUF):
      wait_gather(b)
      compact(b)
      start_gather(_NBUF + b, b)
      start_out(b, b)

    def loop_body(g, carry):
      for b in range(_NBUF):
        ch = g * _NBUF + b
        wait_gather(b)
        wait_out(b)
        compact(b)
        start_gather(ch + _NBUF, b)
        start_out(ch, b)
      return carry

    lax.fori_loop(1, _NGRP - 1, loop_body, 0)
    for b in range(_NBUF):
      ch = (_NGRP - 1) * _NBUF + b
      wait_gather(b)
      wait_out(b)
      compact(b)
      start_out(ch, b)
    for ch in range(_NGRP * _NBUF, _NCH):
      b = ch % _NBUF
      start_gather(ch, b)
      wait_gather(b)
      wait_out(b)
      compact(b)
      start_out(ch, b)
    for b in range(_NBUF):
      wait_out(b)

  return body


_gather = _build()


@jax.jit
def kernel(input, weight):
  tbl = jnp.pad(weight, ((0, 0), (0, _D)))
  idx = input.reshape(_NW, _NCH, _CHUNK).astype(jnp.int32)
  out = _gather(idx, tbl)
  return out.reshape(_B, _H, _D)
